# Initial kernel scaffold; baseline (speedup 1.0000x reference)
import jax
import jax.numpy as jnp
from jax.experimental import pallas as pl


def kernel(x, idx1, mu1, logvar1, bias1, eps1, idx2, mu2, logvar2, bias2, eps2, idx3, mu3, logvar3, bias3, eps3, idx4, mu4, logvar4, bias4, eps4, idx5, mu5, logvar5, bias5, eps5, g1, b1, g2, b2, g3, b3, g4, b4):
    raise NotImplementedError("write your pallas kernel here")


# speedup vs baseline: 1.9356x; 1.9356x over previous
import jax
import jax.numpy as jnp
from jax import lax
from jax.experimental import pallas as pl
from jax.experimental.pallas import tpu as pltpu
from jax.experimental.pallas import tpu_sc as plsc

_B = 200
_BPAD = 208
_NV = _BPAD // 16
_BLK = 128
_NSUB = 16
_NODES = [30000, 10000, 2000, 500, 100, 32]
_EDGES = [90000, 60000, 12000, 2000, 400]


def _lane_bcast(v16, r):
    idx = jnp.full((16,), r, dtype=jnp.int32)
    return jnp.take_along_axis(v16, idx, axis=0)


def _rsqrt16(v):
    i = lax.bitcast_convert_type(v, jnp.int32)
    y = lax.bitcast_convert_type(
        jnp.int32(0x5F3759DF) - lax.shift_right_logical(i, 1), jnp.float32)
    for _ in range(3):
        y = y * (1.5 - 0.5 * v * y * y)
    return y


def _mask8():
    return jnp.arange(16, dtype=jnp.int32) < 8


def _make_level(n_in_rows, n_out, nblk_tile, affine):
    H0 = ((n_out + 31) // 32) * 16
    H1 = n_out - H0
    ACCR = H0 + 16
    NPAD_OUT = H0 + ((H1 + 15) // 16) * 16
    C0 = H0 // 16
    C1 = (H1 + 15) // 16
    NMAX_BN = (C0 + _NSUB - 1) // _NSUB
    CZ = ACCR // 16
    NMAX_Z = (CZ + _NSUB - 1) // _NSUB

    mesh = plsc.VectorSubcoreMesh(core_axis_name="c", subcore_axis_name="s")
    out_type = (jax.ShapeDtypeStruct((NPAD_OUT, _BPAD), jnp.float32),
                jax.ShapeDtypeStruct((16,), jnp.float32))
    scratch = [
        pltpu.VMEM_SHARED((ACCR, _BPAD), jnp.float32),
        pltpu.VMEM_SHARED((_NSUB, 16), jnp.float32),
        pltpu.VMEM((5, _BLK), jnp.float32),
        pltpu.VMEM((_BLK,), jnp.int32),
        pltpu.VMEM((_BLK,), jnp.int32),
        pltpu.VMEM((_BLK,), jnp.float32),
        pltpu.VMEM((_BLK, _BPAD), jnp.float32),
        pltpu.VMEM((16, _BPAD), jnp.float32),
        pltpu.VMEM((16, _BPAD), jnp.float32),
        pltpu.VMEM((16,), jnp.float32),
        pltpu.VMEM((16,), jnp.float32),
        pltpu.VMEM((16,), jnp.float32),
        pltpu.VMEM((_NSUB, 16), jnp.float32),
        pltpu.SemaphoreType.DMA,
    ]

    def body(*refs):
        if affine:
            (x_hbm, edges_hbm, gamma_hbm, beta_hbm, y_hbm, kl_hbm,
             acc, kls, ebuf, src_idx, dst_idx, wbuf, rows, rb, ob,
             g16, b16, klv, klbuf, sem) = refs
        else:
            (x_hbm, edges_hbm, y_hbm, kl_hbm,
             acc, kls, ebuf, src_idx, dst_idx, wbuf, rows, rb, ob,
             g16, b16, klv, klbuf, sem) = refs
        c = lax.axis_index("c")
        s = lax.axis_index("s")
        cbase = c * H0
        hc = jnp.where(c == 0, H0, H1).astype(jnp.int32)
        cchunks = jnp.where(c == 0, C0, C1).astype(jnp.int32)
        zero16 = jnp.zeros((16,), jnp.float32)

        for r in range(16):
            for j in range(_NV):
                rb[r, pl.ds(j * 16, 16)] = zero16

        def zbody(i, carry):
            ch = s + i * _NSUB

            @pl.when(ch < CZ)
            def _():
                pltpu.sync_copy(rb, acc.at[pl.ds(pl.multiple_of(ch * 16, 16),
                                                 16)])
            return carry

        lax.fori_loop(0, NMAX_Z, zbody, 0)
        plsc.subcore_barrier()

        def eblock(b, kl16):
            blk = s * nblk_tile + b
            pltpu.sync_copy(edges_hbm.at[blk], ebuf)
            for k in range(8):
                sl = pl.ds(k * 16, 16)
                src_idx[sl] = lax.bitcast_convert_type(ebuf[0, sl], jnp.int32)
                d = lax.bitcast_convert_type(ebuf[1, sl], jnp.int32)
                dl = d - cbase
                ok = (dl >= 0) & (dl < hc)
                dst_idx[sl] = jnp.where(ok, dl, hc)
                mu = ebuf[2, sl]
                lv = ebuf[3, sl]
                ep = ebuf[4, sl]
                eh = jnp.exp(0.5 * lv)
                wbuf[sl] = mu + eh * ep
                kl16 = kl16 + 0.5 * (mu * mu + eh * eh - lv - 1.0)
            pltpu.async_copy(x_hbm.at[src_idx], rows, sem).wait()

            def scale_body(g, carry):
                base = pl.multiple_of((g // 16) * 16, 16)
                w16 = wbuf[pl.ds(base, 16)]
                wb = _lane_bcast(w16, g - base)
                for j in range(_NV):
                    sj = pl.ds(j * 16, 16)
                    rows[g, sj] = rows[g, sj] * wb
                return carry

            lax.fori_loop(0, _BLK, scale_body, 0)
            pltpu.sync_copy(rows, acc.at[dst_idx], add=True)
            return kl16

        kl16 = lax.fori_loop(0, nblk_tile, eblock, zero16)
        klv[...] = kl16
        pltpu.sync_copy(klv, kls.at[s])
        plsc.subcore_barrier()

        inv_b = jnp.float32(1.0 / _B)
        m8 = _mask8()

        def bnchunk(i, carry):
            ch = s + i * _NSUB

            @pl.when(ch < cchunks)
            def _():
                lo = pl.multiple_of(ch * 16, 16)
                glo = pl.multiple_of(cbase + ch * 16, 16)
                pltpu.sync_copy(acc.at[pl.ds(lo, 16)], rb)
                if affine:
                    pltpu.sync_copy(gamma_hbm.at[pl.ds(glo, 16)], g16)
                    pltpu.sync_copy(beta_hbm.at[pl.ds(glo, 16)], b16)

                def row_body(r, carry2):
                    xs = [rb[r, pl.ds(j * 16, 16)] for j in range(_NV)]
                    xs[12] = jnp.where(m8, xs[12], 0.0)
                    s1 = xs[0]
                    s2 = xs[0] * xs[0]
                    for j in range(1, _NV):
                        s1 = s1 + xs[j]
                        s2 = s2 + xs[j] * xs[j]
                    t1 = jnp.sum(s1)
                    t2 = jnp.sum(s2)
                    m = t1 * inv_b
                    var = t2 * inv_b - m * m
                    inv = _rsqrt16(jnp.full((16,), var + 1e-5, jnp.float32))
                    m16 = jnp.full((16,), m, jnp.float32)
                    if affine:
                        ga = _lane_bcast(g16[...], r)
                        be = _lane_bcast(b16[...], r)
                        sc = inv * ga
                        off = be - m16 * sc
                    else:
                        sc = inv
                        off = -(m16 * inv)
                    for j in range(_NV):
                        y = xs[j] * sc + off
                        if affine:
                            y = jnp.maximum(y, 0.0)
                        if j == 12:
                            y = jnp.where(m8, y, 0.0)
                        ob[r, pl.ds(j * 16, 16)] = y
                    return carry2

                lax.fori_loop(0, 16, row_body, 0)
                pltpu.sync_copy(ob, y_hbm.at[pl.ds(glo, 16)])
            return carry

        lax.fori_loop(0, NMAX_BN, bnchunk, 0)

        @pl.when((c == 0) & (s == 0))
        def _():
            pltpu.sync_copy(kls, klbuf)
            tot = klbuf[0, pl.ds(0, 16)]
            for r in range(1, _NSUB):
                tot = tot + klbuf[r, pl.ds(0, 16)]
            klv[...] = jnp.full((16,), jnp.sum(tot), jnp.float32)
            pltpu.sync_copy(klv, kl_hbm)

    return pl.kernel(
        body, out_type, mesh=mesh, scratch_types=scratch,
        compiler_params=pltpu.CompilerParams(needs_layout_passes=False,
                                             use_tc_tiling_on_sc=False))


def _pack_edges(idx, mu, lv, ep, e_pad):
    e = idx.shape[0]
    pad = e_pad - e
    srcf = lax.bitcast_convert_type(jnp.pad(idx[:, 0], (0, pad)), jnp.float32)
    dstf = lax.bitcast_convert_type(jnp.pad(idx[:, 1], (0, pad)), jnp.float32)
    packed = jnp.stack([srcf, dstf, jnp.pad(mu, (0, pad)),
                        jnp.pad(lv, (0, pad)), jnp.pad(ep, (0, pad))], axis=0)
    return packed.reshape(5, e_pad // _BLK, _BLK).transpose(1, 0, 2)


def _specs():
    specs = []
    n_in_rows = _NODES[0]
    for i in range(5):
        n_out = _NODES[i + 1]
        e_pad = -(-_EDGES[i] // (_NSUB * _BLK)) * (_NSUB * _BLK)
        nblk_tile = e_pad // (_NSUB * _BLK)
        affine = i < 4
        H0 = ((n_out + 31) // 32) * 16
        H1 = n_out - H0
        npad_out = H0 + ((H1 + 15) // 16) * 16
        specs.append(dict(n_in_rows=n_in_rows, n_out=n_out, e_pad=e_pad,
                          nblk_tile=nblk_tile, affine=affine,
                          npad_out=npad_out))
        n_in_rows = npad_out
    return specs


_SPECS = _specs()
_KERNELS = [_make_level(sp["n_in_rows"], sp["n_out"], sp["nblk_tile"],
                        sp["affine"]) for sp in _SPECS]


def kernel(x, idx1, mu1, logvar1, bias1, eps1, idx2, mu2, logvar2, bias2,
           eps2, idx3, mu3, logvar3, bias3, eps3, idx4, mu4, logvar4, bias4,
           eps4, idx5, mu5, logvar5, bias5, eps5, g1, b1, g2, b2, g3, b3,
           g4, b4):
    idxs = [idx1, idx2, idx3, idx4, idx5]
    mus = [mu1, mu2, mu3, mu4, mu5]
    lvs = [logvar1, logvar2, logvar3, logvar4, logvar5]
    eps = [eps1, eps2, eps3, eps4, eps5]
    gammas = [g1, g2, g3, g4, None]
    betas = [b1, b2, b3, b4, None]

    cur = jnp.pad(jnp.transpose(x.reshape(_B, -1)), ((0, 0), (0, _BPAD - _B)))
    kl_tot = jnp.float32(0.0)
    for i in range(5):
        sp = _SPECS[i]
        packed = _pack_edges(idxs[i], mus[i], lvs[i], eps[i], sp["e_pad"])
        if sp["affine"]:
            gp = jnp.pad(gammas[i], (0, sp["npad_out"] - sp["n_out"]))
            bp = jnp.pad(betas[i], (0, sp["npad_out"] - sp["n_out"]))
            cur, kl16 = _KERNELS[i](cur, packed, gp, bp)
        else:
            cur, kl16 = _KERNELS[i](cur, packed)
        kl_tot = kl_tot + kl16[0]
    y = jnp.transpose(cur[:, :_B])
    return y, kl_tot


# speedup vs baseline: 1.9379x; 1.0012x over previous
import jax
import jax.numpy as jnp
from jax import lax
from jax.experimental import pallas as pl
from jax.experimental.pallas import tpu as pltpu
from jax.experimental.pallas import tpu_sc as plsc

_B = 200
_BPAD = 208
_NV = _BPAD // 16
_BLK = 128
_NSUB = 16
_NODES = [30000, 10000, 2000, 500, 100, 32]
_EDGES = [90000, 60000, 12000, 2000, 400]


def _lane_bcast(v16, r):
    idx = jnp.full((16,), r, dtype=jnp.int32)
    return jnp.take_along_axis(v16, idx, axis=0)


def _rsqrt16(v):
    i = lax.bitcast_convert_type(v, jnp.int32)
    y = lax.bitcast_convert_type(
        jnp.int32(0x5F3759DF) - lax.shift_right_logical(i, 1), jnp.float32)
    for _ in range(3):
        y = y * (1.5 - 0.5 * v * y * y)
    return y


def _mask8():
    return jnp.arange(16, dtype=jnp.int32) < 8


def _make_level(n_in_rows, n_out, nblk_tile, affine):
    H0 = ((n_out + 31) // 32) * 16
    H1 = n_out - H0
    ACCR = H0 + 16
    NPAD_OUT = H0 + ((H1 + 15) // 16) * 16
    C0 = H0 // 16
    C1 = (H1 + 15) // 16
    NMAX_BN = (C0 + _NSUB - 1) // _NSUB
    CZ = ACCR // 16
    NMAX_Z = (CZ + _NSUB - 1) // _NSUB

    mesh = plsc.VectorSubcoreMesh(core_axis_name="c", subcore_axis_name="s")
    out_type = (jax.ShapeDtypeStruct((NPAD_OUT, _BPAD), jnp.float32),
                jax.ShapeDtypeStruct((16,), jnp.float32))
    scratch = [
        pltpu.VMEM_SHARED((ACCR, _BPAD), jnp.float32),
        pltpu.VMEM_SHARED((_NSUB, 16), jnp.float32),
        pltpu.VMEM((5, _BLK), jnp.float32),
        pltpu.VMEM((_BLK,), jnp.int32),
        pltpu.VMEM((_BLK,), jnp.int32),
        pltpu.VMEM((_BLK,), jnp.float32),
        pltpu.VMEM((_BLK, _BPAD), jnp.float32),
        pltpu.VMEM((16, _BPAD), jnp.float32),
        pltpu.VMEM((16, _BPAD), jnp.float32),
        pltpu.VMEM((16,), jnp.float32),
        pltpu.VMEM((16,), jnp.float32),
        pltpu.VMEM((16,), jnp.float32),
        pltpu.VMEM((_NSUB, 16), jnp.float32),
        pltpu.SemaphoreType.DMA,
    ]

    def body(*refs):
        if affine:
            (x_hbm, edges_hbm, gamma_hbm, beta_hbm, y_hbm, kl_hbm,
             acc, kls, ebuf, src_idx, dst_idx, wbuf, rows, rb, ob,
             g16, b16, klv, klbuf, sem) = refs
        else:
            (x_hbm, edges_hbm, y_hbm, kl_hbm,
             acc, kls, ebuf, src_idx, dst_idx, wbuf, rows, rb, ob,
             g16, b16, klv, klbuf, sem) = refs
        c = lax.axis_index("c")
        s = lax.axis_index("s")
        cbase = c * H0
        hc = jnp.where(c == 0, H0, H1).astype(jnp.int32)
        trash = hc + s
        cchunks = jnp.where(c == 0, C0, C1).astype(jnp.int32)
        zero16 = jnp.zeros((16,), jnp.float32)

        for r in range(16):
            for j in range(_NV):
                rb[r, pl.ds(j * 16, 16)] = zero16

        def zbody(i, carry):
            ch = s + i * _NSUB

            @pl.when(ch < CZ)
            def _():
                pltpu.sync_copy(rb, acc.at[pl.ds(pl.multiple_of(ch * 16, 16),
                                                 16)])
            return carry

        lax.fori_loop(0, NMAX_Z, zbody, 0)
        plsc.subcore_barrier()

        def eblock(b, kl16):
            blk = s * nblk_tile + b
            pltpu.sync_copy(edges_hbm.at[blk], ebuf)
            for k in range(8):
                sl = pl.ds(k * 16, 16)
                src_idx[sl] = lax.bitcast_convert_type(ebuf[0, sl], jnp.int32)
                d = lax.bitcast_convert_type(ebuf[1, sl], jnp.int32)
                dl = d - cbase
                ok = (dl >= 0) & (dl < hc)
                dst_idx[sl] = jnp.where(ok, dl, trash)
                mu = ebuf[2, sl]
                lv = ebuf[3, sl]
                ep = ebuf[4, sl]
                eh = jnp.exp(0.5 * lv)
                wbuf[sl] = mu + eh * ep
                kl16 = kl16 + 0.5 * (mu * mu + eh * eh - lv - 1.0)
            pltpu.async_copy(x_hbm.at[src_idx], rows, sem).wait()

            def scale_body(g, carry):
                base = pl.multiple_of((g // 16) * 16, 16)
                w16 = wbuf[pl.ds(base, 16)]
                wb = _lane_bcast(w16, g - base)
                for j in range(_NV):
                    sj = pl.ds(j * 16, 16)
                    rows[g, sj] = rows[g, sj] * wb
                return carry

            lax.fori_loop(0, _BLK, scale_body, 0)
            pltpu.sync_copy(rows, acc.at[dst_idx], add=True)
            return kl16

        kl16 = lax.fori_loop(0, nblk_tile, eblock, zero16)
        klv[...] = kl16
        pltpu.sync_copy(klv, kls.at[s])
        plsc.subcore_barrier()

        inv_b = jnp.float32(1.0 / _B)
        m8 = _mask8()

        def bnchunk(i, carry):
            ch = s + i * _NSUB

            @pl.when(ch < cchunks)
            def _():
                lo = pl.multiple_of(ch * 16, 16)
                glo = pl.multiple_of(cbase + ch * 16, 16)
                pltpu.sync_copy(acc.at[pl.ds(lo, 16)], rb)
                if affine:
                    pltpu.sync_copy(gamma_hbm.at[pl.ds(glo, 16)], g16)
                    pltpu.sync_copy(beta_hbm.at[pl.ds(glo, 16)], b16)

                def row_body(r, carry2):
                    xs = [rb[r, pl.ds(j * 16, 16)] for j in range(_NV)]
                    xs[12] = jnp.where(m8, xs[12], 0.0)
                    s1 = xs[0]
                    s2 = xs[0] * xs[0]
                    for j in range(1, _NV):
                        s1 = s1 + xs[j]
                        s2 = s2 + xs[j] * xs[j]
                    t1 = jnp.sum(s1)
                    t2 = jnp.sum(s2)
                    m = t1 * inv_b
                    var = t2 * inv_b - m * m
                    inv = _rsqrt16(jnp.full((16,), var + 1e-5, jnp.float32))
                    m16 = jnp.full((16,), m, jnp.float32)
                    if affine:
                        ga = _lane_bcast(g16[...], r)
                        be = _lane_bcast(b16[...], r)
                        sc = inv * ga
                        off = be - m16 * sc
                    else:
                        sc = inv
                        off = -(m16 * inv)
                    for j in range(_NV):
                        y = xs[j] * sc + off
                        if affine:
                            y = jnp.maximum(y, 0.0)
                        if j == 12:
                            y = jnp.where(m8, y, 0.0)
                        ob[r, pl.ds(j * 16, 16)] = y
                    return carry2

                lax.fori_loop(0, 16, row_body, 0)
                pltpu.sync_copy(ob, y_hbm.at[pl.ds(glo, 16)])
            return carry

        lax.fori_loop(0, NMAX_BN, bnchunk, 0)

        @pl.when((c == 0) & (s == 0))
        def _():
            pltpu.sync_copy(kls, klbuf)
            tot = klbuf[0, pl.ds(0, 16)]
            for r in range(1, _NSUB):
                tot = tot + klbuf[r, pl.ds(0, 16)]
            klv[...] = jnp.full((16,), jnp.sum(tot), jnp.float32)
            pltpu.sync_copy(klv, kl_hbm)

    return pl.kernel(
        body, out_type, mesh=mesh, scratch_types=scratch,
        compiler_params=pltpu.CompilerParams(needs_layout_passes=False,
                                             use_tc_tiling_on_sc=False))


def _pack_edges(idx, mu, lv, ep, e_pad):
    e = idx.shape[0]
    pad = e_pad - e
    srcf = lax.bitcast_convert_type(jnp.pad(idx[:, 0], (0, pad)), jnp.float32)
    dstf = lax.bitcast_convert_type(
        jnp.pad(idx[:, 1], (0, pad), constant_values=1 << 30), jnp.float32)
    packed = jnp.stack([srcf, dstf, jnp.pad(mu, (0, pad)),
                        jnp.pad(lv, (0, pad)), jnp.pad(ep, (0, pad))], axis=0)
    return packed.reshape(5, e_pad // _BLK, _BLK).transpose(1, 0, 2)


def _specs():
    specs = []
    n_in_rows = _NODES[0]
    for i in range(5):
        n_out = _NODES[i + 1]
        e_pad = -(-_EDGES[i] // (_NSUB * _BLK)) * (_NSUB * _BLK)
        nblk_tile = e_pad // (_NSUB * _BLK)
        affine = i < 4
        H0 = ((n_out + 31) // 32) * 16
        H1 = n_out - H0
        npad_out = H0 + ((H1 + 15) // 16) * 16
        specs.append(dict(n_in_rows=n_in_rows, n_out=n_out, e_pad=e_pad,
                          nblk_tile=nblk_tile, affine=affine,
                          npad_out=npad_out))
        n_in_rows = npad_out
    return specs


_SPECS = _specs()
_KERNELS = [_make_level(sp["n_in_rows"], sp["n_out"], sp["nblk_tile"],
                        sp["affine"]) for sp in _SPECS]


def kernel(x, idx1, mu1, logvar1, bias1, eps1, idx2, mu2, logvar2, bias2,
           eps2, idx3, mu3, logvar3, bias3, eps3, idx4, mu4, logvar4, bias4,
           eps4, idx5, mu5, logvar5, bias5, eps5, g1, b1, g2, b2, g3, b3,
           g4, b4):
    idxs = [idx1, idx2, idx3, idx4, idx5]
    mus = [mu1, mu2, mu3, mu4, mu5]
    lvs = [logvar1, logvar2, logvar3, logvar4, logvar5]
    eps = [eps1, eps2, eps3, eps4, eps5]
    gammas = [g1, g2, g3, g4, None]
    betas = [b1, b2, b3, b4, None]

    cur = jnp.pad(jnp.transpose(x.reshape(_B, -1)), ((0, 0), (0, _BPAD - _B)))
    kl_tot = jnp.float32(0.0)
    for i in range(5):
        sp = _SPECS[i]
        packed = _pack_edges(idxs[i], mus[i], lvs[i], eps[i], sp["e_pad"])
        if sp["affine"]:
            gp = jnp.pad(gammas[i], (0, sp["npad_out"] - sp["n_out"]))
            bp = jnp.pad(betas[i], (0, sp["npad_out"] - sp["n_out"]))
            cur, kl16 = _KERNELS[i](cur, packed, gp, bp)
        else:
            cur, kl16 = _KERNELS[i](cur, packed)
        kl_tot = kl_tot + kl16[0]
    y = jnp.transpose(cur[:, :_B])
    return y, kl_tot


# speedup vs baseline: 2.1462x; 1.1075x over previous
import jax
import jax.numpy as jnp
from jax import lax
from jax.experimental import pallas as pl
from jax.experimental.pallas import tpu as pltpu
from jax.experimental.pallas import tpu_sc as plsc

_B = 200
_BPAD = 208
_NV = _BPAD // 16
_BLK = 128
_NSUB = 16
_NODES = [30000, 10000, 2000, 500, 100, 32]
_EDGES = [90000, 60000, 12000, 2000, 400]


def _lane_bcast(v16, r):
    idx = jnp.full((16,), r, dtype=jnp.int32)
    return jnp.take_along_axis(v16, idx, axis=0)


def _rsqrt16(v):
    i = lax.bitcast_convert_type(v, jnp.int32)
    y = lax.bitcast_convert_type(
        jnp.int32(0x5F3759DF) - lax.shift_right_logical(i, 1), jnp.float32)
    for _ in range(3):
        y = y * (1.5 - 0.5 * v * y * y)
    return y


def _mask8():
    return jnp.arange(16, dtype=jnp.int32) < 8


def _stg(nblk_tile):
    for d in (8, 7, 6, 5, 4, 3, 2, 1):
        if nblk_tile % d == 0:
            return d
    return 1


def _make_level(n_in_rows, n_out, nblk_tile, affine):
    H0 = ((n_out + 31) // 32) * 16
    H1 = n_out - H0
    NPASS = 2 if n_out >= 4000 else 1
    Q = -(-H0 // (16 * NPASS)) * 16
    ACCR = Q + 16
    NPAD_OUT = H0 + ((H1 + 15) // 16) * 16
    NMAX_BN = (Q // 16 + _NSUB - 1) // _NSUB
    CZ = ACCR // 16
    NMAX_Z = (CZ + _NSUB - 1) // _NSUB
    STG = _stg(nblk_tile)
    NSTAGE = nblk_tile // STG
    NG = nblk_tile
    CAP = NG * _BLK + 16

    mesh = plsc.VectorSubcoreMesh(core_axis_name="c", subcore_axis_name="s")
    out_type = (jax.ShapeDtypeStruct((NPAD_OUT, _BPAD), jnp.float32),
                jax.ShapeDtypeStruct((16,), jnp.float32))
    scratch = [
        pltpu.VMEM_SHARED((ACCR, _BPAD), jnp.float32),
        pltpu.VMEM_SHARED((_NSUB, 16), jnp.float32),
        pltpu.VMEM((STG, 5, _BLK), jnp.float32),
        pltpu.VMEM((STG, 5, _BLK), jnp.float32),
        pltpu.VMEM((CAP,), jnp.int32),
        pltpu.VMEM((CAP,), jnp.int32),
        pltpu.VMEM((CAP,), jnp.float32),
        pltpu.VMEM((NG, _BLK), jnp.int32),
        pltpu.VMEM((NG, _BLK), jnp.int32),
        pltpu.VMEM((_BLK, _BPAD), jnp.float32),
        pltpu.VMEM((_BLK, _BPAD), jnp.float32),
        pltpu.VMEM((16, _BPAD), jnp.float32),
        pltpu.VMEM((16, _BPAD), jnp.float32),
        pltpu.VMEM((16,), jnp.float32),
        pltpu.VMEM((16,), jnp.float32),
        pltpu.VMEM((16,), jnp.float32),
        pltpu.VMEM((_NSUB, 16), jnp.float32),
        pltpu.SemaphoreType.DMA,
        pltpu.SemaphoreType.DMA,
        pltpu.SemaphoreType.DMA,
        pltpu.SemaphoreType.DMA,
        pltpu.SemaphoreType.DMA,
        pltpu.SemaphoreType.DMA,
    ]

    def body(*refs):
        if affine:
            (x_hbm, edges_hbm, gamma_hbm, beta_hbm, y_hbm, kl_hbm,
             acc, kls, ebufA, ebufB, csrc_tmp, cdl_tmp, cw, csrc2, cdl2,
             rowsA, rowsB, rb, ob, g16, b16, klv, klbuf,
             semEA, semEB, semGA, semGB, semSA, semSB) = refs
        else:
            (x_hbm, edges_hbm, y_hbm, kl_hbm,
             acc, kls, ebufA, ebufB, csrc_tmp, cdl_tmp, cw, csrc2, cdl2,
             rowsA, rowsB, rb, ob, g16, b16, klv, klbuf,
             semEA, semEB, semGA, semGB, semSA, semSB) = refs
        c = lax.axis_index("c")
        s = lax.axis_index("s")
        cbase = c * H0
        hc = jnp.where(c == 0, H0, H1).astype(jnp.int32)
        zero16 = jnp.zeros((16,), jnp.float32)
        lanes = jnp.arange(16, dtype=jnp.int32)
        tbase = s * nblk_tile
        inv_b = jnp.float32(1.0 / _B)
        m8 = _mask8()

        for r in range(16):
            for j in range(_NV):
                rb[r, pl.ds(j * 16, 16)] = zero16

        for p in range(NPASS):
            pbase = cbase + p * Q
            hcp = jnp.clip(hc - p * Q, 0, Q)
            trash = hcp + s
            cchunks = (hcp + 15) // 16

            def zbody(i, carry):
                ch = s + i * _NSUB

                @pl.when(ch < CZ)
                def _():
                    pltpu.sync_copy(
                        rb, acc.at[pl.ds(pl.multiple_of(ch * 16, 16), 16)])
                return carry

            lax.fori_loop(0, NMAX_Z, zbody, 0)
            plsc.subcore_barrier()

            def estage_start(t, ebuf, sem):
                src = edges_hbm.at[pl.ds(tbase + t * STG, STG)]
                pltpu.async_copy(src, ebuf, sem)

            def estage_wait(ebuf, sem):
                pltpu.make_async_copy(edges_hbm.at[pl.ds(0, STG)], ebuf,
                                      sem).wait()

            def estage_proc(ebuf, carry):
                cnt, kl16 = carry
                for q in range(STG):
                    for k in range(8):
                        sl = pl.ds(k * 16, 16)
                        src_i = lax.bitcast_convert_type(ebuf[q, 0, sl],
                                                         jnp.int32)
                        d = lax.bitcast_convert_type(ebuf[q, 1, sl],
                                                     jnp.int32)
                        dl = d - pbase
                        ok = (dl >= 0) & (dl < hcp)
                        lv = ebuf[q, 3, sl]
                        mu = ebuf[q, 2, sl]
                        eh = jnp.exp(0.5 * lv)
                        w = mu + eh * ebuf[q, 4, sl]
                        if p == 0:
                            kl16 = kl16 + 0.5 * (mu * mu + eh * eh - lv
                                                 - 1.0)
                        plsc.store_compressed(csrc_tmp.at[pl.ds(cnt, 16)],
                                              src_i, mask=ok)
                        plsc.store_compressed(cdl_tmp.at[pl.ds(cnt, 16)],
                                              dl, mask=ok)
                        plsc.store_compressed(cw.at[pl.ds(cnt, 16)], w,
                                              mask=ok)
                        cnt = cnt + jnp.sum(
                            jnp.where(ok, 1.0, 0.0)).astype(jnp.int32)
                return cnt, kl16

            estage_start(0, ebufA, semEA)

            def estage_pair(pp, carry):
                t1 = 2 * pp + 1

                @pl.when(t1 < NSTAGE)
                def _():
                    estage_start(t1, ebufB, semEB)
                estage_wait(ebufA, semEA)
                carry = estage_proc(ebufA, carry)

                def odd(carry):
                    @pl.when(t1 + 1 < NSTAGE)
                    def _():
                        estage_start(t1 + 1, ebufA, semEA)
                    estage_wait(ebufB, semEB)
                    return estage_proc(ebufB, carry)

                carry = lax.cond(t1 < NSTAGE, odd, lambda cr: cr, carry)
                return carry

            cnt, kl16 = lax.fori_loop(0, (NSTAGE + 1) // 2, estage_pair,
                                      (jnp.int32(0), zero16))
            if p == 0:
                klv[...] = kl16
                pltpu.sync_copy(klv, kls.at[s])

            m = cnt
            ngroups = (m + _BLK - 1) // _BLK
            mround = ngroups * _BLK
            mfloor = pl.multiple_of((m // 16) * 16, 16)

            def padv(t, carry):
                base = mfloor + t * 16

                @pl.when(base < mround)
                def _():
                    sl = pl.ds(base, 16)
                    lm = (lanes + base) >= m
                    csrc_tmp[sl] = jnp.where(lm, 0, csrc_tmp[sl])
                    cdl_tmp[sl] = jnp.where(lm, trash, cdl_tmp[sl])
                    cw[sl] = jnp.where(lm, 0.0, cw[sl])
                return carry

            lax.fori_loop(0, 9, padv, 0)

            def cpy(i, carry):
                g = i // 8
                o = pl.multiple_of((i % 8) * 16, 16)
                sl = pl.ds(pl.multiple_of(i * 16, 16), 16)
                csrc2[g, pl.ds(o, 16)] = csrc_tmp[sl]
                cdl2[g, pl.ds(o, 16)] = cdl_tmp[sl]
                return carry

            lax.fori_loop(0, ngroups * 8, cpy, 0)

            def g_start(g, rows, sem):
                pltpu.async_copy(x_hbm.at[csrc2.at[g]], rows, sem)

            def g_wait(rows, sem):
                pltpu.make_async_copy(x_hbm.at[csrc2.at[0]], rows,
                                      sem).wait()

            def s_start(g, rows, sem):
                pltpu.async_copy(rows, acc.at[cdl2.at[g]], sem, add=True)

            def s_wait(rows, sem):
                pltpu.make_async_copy(rows, acc.at[cdl2.at[0]], sem).wait()

            def scale(g, rows):
                goff = pl.multiple_of(g * _BLK, _BLK)

                def scale_body(r, carry):
                    wbase = pl.multiple_of((r // 16) * 16, 16)
                    w16 = cw[pl.ds(goff + wbase, 16)]
                    wb = _lane_bcast(w16, r - wbase)
                    for j in range(_NV):
                        sj = pl.ds(j * 16, 16)
                        rows[r, sj] = rows[r, sj] * wb
                    return carry

                lax.fori_loop(0, _BLK, scale_body, 0)

            @pl.when(ngroups > 0)
            def _():
                g_start(0, rowsA, semGA)

            def gpair(pp, carry):
                g0 = 2 * pp
                g1 = 2 * pp + 1

                @pl.when(g0 < ngroups)
                def _():
                    @pl.when(g1 < ngroups)
                    def _():
                        @pl.when(g1 >= 2)
                        def _():
                            s_wait(rowsB, semSB)
                        g_start(g1, rowsB, semGB)
                    g_wait(rowsA, semGA)
                    scale(g0, rowsA)
                    s_start(g0, rowsA, semSA)

                @pl.when(g1 < ngroups)
                def _():
                    @pl.when(g1 + 1 < ngroups)
                    def _():
                        @pl.when(g1 + 1 >= 2)
                        def _():
                            s_wait(rowsA, semSA)
                        g_start(g1 + 1, rowsA, semGA)
                    g_wait(rowsB, semGB)
                    scale(g1, rowsB)
                    s_start(g1, rowsB, semSB)
                return carry

            lax.fori_loop(0, (NG + 1) // 2, gpair, 0)
            odd = (ngroups % 2) == 1

            @pl.when((ngroups >= 1) & odd)
            def _():
                s_wait(rowsA, semSA)

            @pl.when((ngroups >= 1) & ~odd)
            def _():
                s_wait(rowsB, semSB)

            @pl.when((ngroups >= 2) & odd)
            def _():
                s_wait(rowsB, semSB)

            @pl.when((ngroups >= 2) & ~odd)
            def _():
                s_wait(rowsA, semSA)

            plsc.subcore_barrier()

            def bnchunk(i, carry):
                ch = s + i * _NSUB

                @pl.when(ch < cchunks)
                def _():
                    lo = pl.multiple_of(ch * 16, 16)
                    glo = pl.multiple_of(pbase + ch * 16, 16)
                    pltpu.sync_copy(acc.at[pl.ds(lo, 16)], rb)
                    if affine:
                        pltpu.sync_copy(gamma_hbm.at[pl.ds(glo, 16)], g16)
                        pltpu.sync_copy(beta_hbm.at[pl.ds(glo, 16)], b16)

                    def row_body(r, carry2):
                        xs = [rb[r, pl.ds(j * 16, 16)] for j in range(_NV)]
                        xs[12] = jnp.where(m8, xs[12], 0.0)
                        s1 = xs[0]
                        s2 = xs[0] * xs[0]
                        for j in range(1, _NV):
                            s1 = s1 + xs[j]
                            s2 = s2 + xs[j] * xs[j]
                        t1 = jnp.sum(s1)
                        t2 = jnp.sum(s2)
                        mm = t1 * inv_b
                        var = t2 * inv_b - mm * mm
                        inv = _rsqrt16(jnp.full((16,), var + 1e-5,
                                                jnp.float32))
                        m16 = jnp.full((16,), mm, jnp.float32)
                        if affine:
                            ga = _lane_bcast(g16[...], r)
                            be = _lane_bcast(b16[...], r)
                            sc = inv * ga
                            off = be - m16 * sc
                        else:
                            sc = inv
                            off = -(m16 * inv)
                        for j in range(_NV):
                            y = xs[j] * sc + off
                            if affine:
                                y = jnp.maximum(y, 0.0)
                            if j == 12:
                                y = jnp.where(m8, y, 0.0)
                            ob[r, pl.ds(j * 16, 16)] = y
                        return carry2

                    lax.fori_loop(0, 16, row_body, 0)
                    pltpu.sync_copy(ob, y_hbm.at[pl.ds(glo, 16)])
                return carry

            lax.fori_loop(0, NMAX_BN, bnchunk, 0)

            if p + 1 < NPASS:
                for r in range(16):
                    for j in range(_NV):
                        rb[r, pl.ds(j * 16, 16)] = zero16
                plsc.subcore_barrier()

        @pl.when((c == 0) & (s == 0))
        def _():
            pltpu.sync_copy(kls, klbuf)
            tot = klbuf[0, pl.ds(0, 16)]
            for r in range(1, _NSUB):
                tot = tot + klbuf[r, pl.ds(0, 16)]
            klv[...] = jnp.full((16,), jnp.sum(tot), jnp.float32)
            pltpu.sync_copy(klv, kl_hbm)

    return pl.kernel(
        body, out_type, mesh=mesh, scratch_types=scratch,
        compiler_params=pltpu.CompilerParams(needs_layout_passes=False,
                                             use_tc_tiling_on_sc=False))


def _pack_edges(idx, mu, lv, ep, e_pad):
    e = idx.shape[0]
    pad = e_pad - e
    srcf = lax.bitcast_convert_type(jnp.pad(idx[:, 0], (0, pad)), jnp.float32)
    dstf = lax.bitcast_convert_type(
        jnp.pad(idx[:, 1], (0, pad), constant_values=1 << 30), jnp.float32)
    packed = jnp.stack([srcf, dstf, jnp.pad(mu, (0, pad)),
                        jnp.pad(lv, (0, pad)), jnp.pad(ep, (0, pad))], axis=0)
    return packed.reshape(5, e_pad // _BLK, _BLK).transpose(1, 0, 2)


def _specs():
    specs = []
    n_in_rows = _NODES[0]
    for i in range(5):
        n_out = _NODES[i + 1]
        e_pad = -(-_EDGES[i] // (_NSUB * _BLK)) * (_NSUB * _BLK)
        nblk_tile = e_pad // (_NSUB * _BLK)
        affine = i < 4
        H0 = ((n_out + 31) // 32) * 16
        H1 = n_out - H0
        npad_out = H0 + ((H1 + 15) // 16) * 16
        specs.append(dict(n_in_rows=n_in_rows, n_out=n_out, e_pad=e_pad,
                          nblk_tile=nblk_tile, affine=affine,
                          npad_out=npad_out))
        n_in_rows = npad_out
    return specs


_SPECS = _specs()
_KERNELS = [_make_level(sp["n_in_rows"], sp["n_out"], sp["nblk_tile"],
                        sp["affine"]) for sp in _SPECS]


def kernel(x, idx1, mu1, logvar1, bias1, eps1, idx2, mu2, logvar2, bias2,
           eps2, idx3, mu3, logvar3, bias3, eps3, idx4, mu4, logvar4, bias4,
           eps4, idx5, mu5, logvar5, bias5, eps5, g1, b1, g2, b2, g3, b3,
           g4, b4):
    idxs = [idx1, idx2, idx3, idx4, idx5]
    mus = [mu1, mu2, mu3, mu4, mu5]
    lvs = [logvar1, logvar2, logvar3, logvar4, logvar5]
    eps = [eps1, eps2, eps3, eps4, eps5]
    gammas = [g1, g2, g3, g4, None]
    betas = [b1, b2, b3, b4, None]

    cur = jnp.pad(jnp.transpose(x.reshape(_B, -1)), ((0, 0), (0, _BPAD - _B)))
    kl_tot = jnp.float32(0.0)
    for i in range(5):
        sp = _SPECS[i]
        packed = _pack_edges(idxs[i], mus[i], lvs[i], eps[i], sp["e_pad"])
        if sp["affine"]:
            gp = jnp.pad(gammas[i], (0, sp["npad_out"] - sp["n_out"]))
            bp = jnp.pad(betas[i], (0, sp["npad_out"] - sp["n_out"]))
            cur, kl16 = _KERNELS[i](cur, packed, gp, bp)
        else:
            cur, kl16 = _KERNELS[i](cur, packed)
        kl_tot = kl_tot + kl16[0]
    y = jnp.transpose(cur[:, :_B])
    return y, kl_tot


# speedup vs baseline: 2.4866x; 1.1586x over previous
import jax
import jax.numpy as jnp
from jax import lax
from jax.experimental import pallas as pl
from jax.experimental.pallas import tpu as pltpu
from jax.experimental.pallas import tpu_sc as plsc

_B = 200
_BPAD = 208
_NV = _BPAD // 16
_BLK = 128
_NSUB = 16
_NODES = [30000, 10000, 2000, 500, 100, 32]
_EDGES = [90000, 60000, 12000, 2000, 400]


def _lane_bcast(v16, r):
    idx = jnp.full((16,), r, dtype=jnp.int32)
    return jnp.take_along_axis(v16, idx, axis=0)


def _rsqrt16(v):
    i = lax.bitcast_convert_type(v, jnp.int32)
    y = lax.bitcast_convert_type(
        jnp.int32(0x5F3759DF) - lax.shift_right_logical(i, 1), jnp.float32)
    for _ in range(3):
        y = y * (1.5 - 0.5 * v * y * y)
    return y


def _mask8():
    return jnp.arange(16, dtype=jnp.int32) < 8


def _stg(nblk_tile):
    for d in (8, 7, 6, 5, 4, 3, 2, 1):
        if nblk_tile % d == 0:
            return d
    return 1


def _make_level(n_in_rows, n_out, nblk_tile, affine):
    H0 = ((n_out + 31) // 32) * 16
    H1 = n_out - H0
    NPASS = 2 if n_out >= 4000 else 1
    Q = -(-H0 // (16 * NPASS)) * 16
    ACCR = Q + 16
    NPAD_OUT = H0 + ((H1 + 15) // 16) * 16
    NMAX_BN = (Q // 16 + _NSUB - 1) // _NSUB
    CZ = ACCR // 16
    NMAX_Z = (CZ + _NSUB - 1) // _NSUB
    STG = _stg(nblk_tile)
    NSTAGE = nblk_tile // STG
    NG = nblk_tile
    CAP = NG * _BLK + 16

    mesh = plsc.VectorSubcoreMesh(core_axis_name="c", subcore_axis_name="s")
    out_type = (jax.ShapeDtypeStruct((NPAD_OUT, _BPAD), jnp.float32),
                jax.ShapeDtypeStruct((16,), jnp.float32))
    scratch = [
        pltpu.VMEM_SHARED((ACCR, _BPAD), jnp.float32),
        pltpu.VMEM_SHARED((_NSUB, 16), jnp.float32),
        pltpu.VMEM((STG, 5, _BLK), jnp.float32),
        pltpu.VMEM((STG, 5, _BLK), jnp.float32),
        pltpu.VMEM((CAP,), jnp.int32),
        pltpu.VMEM((CAP,), jnp.int32),
        pltpu.VMEM((CAP,), jnp.float32),
        pltpu.VMEM((NG, _BLK), jnp.int32),
        pltpu.VMEM((NG, _BLK), jnp.int32),
        pltpu.VMEM((_BLK, _BPAD), jnp.float32),
        pltpu.VMEM((_BLK, _BPAD), jnp.float32),
        pltpu.VMEM((16, _BPAD), jnp.float32),
        pltpu.VMEM((16, _BPAD), jnp.float32),
        pltpu.VMEM((16,), jnp.float32),
        pltpu.VMEM((16,), jnp.float32),
        pltpu.VMEM((16,), jnp.float32),
        pltpu.VMEM((_NSUB, 16), jnp.float32),
        pltpu.SemaphoreType.DMA,
        pltpu.SemaphoreType.DMA,
        pltpu.SemaphoreType.DMA,
        pltpu.SemaphoreType.DMA,
        pltpu.SemaphoreType.DMA,
        pltpu.SemaphoreType.DMA,
    ]

    def body(*refs):
        if affine:
            (x_hbm, edges_hbm, gamma_hbm, beta_hbm, y_hbm, kl_hbm,
             acc, kls, ebufA, ebufB, csrc_tmp, cdl_tmp, cw, csrc2, cdl2,
             rowsA, rowsB, rb, ob, g16, b16, klv, klbuf,
             semEA, semEB, semGA, semGB, semSA, semSB) = refs
        else:
            (x_hbm, edges_hbm, y_hbm, kl_hbm,
             acc, kls, ebufA, ebufB, csrc_tmp, cdl_tmp, cw, csrc2, cdl2,
             rowsA, rowsB, rb, ob, g16, b16, klv, klbuf,
             semEA, semEB, semGA, semGB, semSA, semSB) = refs
        c = lax.axis_index("c")
        s = lax.axis_index("s")
        cbase = c * H0
        hc = jnp.where(c == 0, H0, H1).astype(jnp.int32)
        zero16 = jnp.zeros((16,), jnp.float32)
        lanes = jnp.arange(16, dtype=jnp.int32)
        tbase = s * nblk_tile
        inv_b = jnp.float32(1.0 / _B)
        m8 = _mask8()

        for r in range(16):
            for j in range(_NV):
                rb[r, pl.ds(j * 16, 16)] = zero16

        for p in range(NPASS):
            pbase = cbase + p * Q
            hcp = jnp.clip(hc - p * Q, 0, Q)
            trash = hcp + s
            cchunks = (hcp + 15) // 16

            def zbody(i, carry):
                ch = s + i * _NSUB

                @pl.when(ch < CZ)
                def _():
                    pltpu.sync_copy(
                        rb, acc.at[pl.ds(pl.multiple_of(ch * 16, 16), 16)])
                return carry

            lax.fori_loop(0, NMAX_Z, zbody, 0)
            plsc.subcore_barrier()

            def estage_start(t, ebuf, sem):
                src = edges_hbm.at[pl.ds(tbase + t * STG, STG)]
                pltpu.async_copy(src, ebuf, sem)

            def estage_wait(ebuf, sem):
                pltpu.make_async_copy(edges_hbm.at[pl.ds(0, STG)], ebuf,
                                      sem).wait()

            def estage_proc(ebuf, carry):
                cnt, kl16 = carry
                for q in range(STG):
                    for k in range(8):
                        sl = pl.ds(k * 16, 16)
                        src_i = lax.bitcast_convert_type(ebuf[q, 0, sl],
                                                         jnp.int32)
                        d = lax.bitcast_convert_type(ebuf[q, 1, sl],
                                                     jnp.int32)
                        dl = d - pbase
                        ok = (dl >= 0) & (dl < hcp)
                        lv = ebuf[q, 3, sl]
                        mu = ebuf[q, 2, sl]
                        eh = jnp.exp(0.5 * lv)
                        w = mu + eh * ebuf[q, 4, sl]
                        if p == 0:
                            kl16 = kl16 + 0.5 * (mu * mu + eh * eh - lv
                                                 - 1.0)
                        plsc.store_compressed(csrc_tmp.at[pl.ds(cnt, 16)],
                                              src_i, mask=ok)
                        plsc.store_compressed(cdl_tmp.at[pl.ds(cnt, 16)],
                                              dl, mask=ok)
                        plsc.store_compressed(cw.at[pl.ds(cnt, 16)], w,
                                              mask=ok)
                        cnt = cnt + jnp.sum(
                            jnp.where(ok, 1.0, 0.0)).astype(jnp.int32)
                return cnt, kl16

            estage_start(0, ebufA, semEA)

            def estage_pair(pp, carry):
                t1 = 2 * pp + 1

                @pl.when(t1 < NSTAGE)
                def _():
                    estage_start(t1, ebufB, semEB)
                estage_wait(ebufA, semEA)
                carry = estage_proc(ebufA, carry)

                def odd(carry):
                    @pl.when(t1 + 1 < NSTAGE)
                    def _():
                        estage_start(t1 + 1, ebufA, semEA)
                    estage_wait(ebufB, semEB)
                    return estage_proc(ebufB, carry)

                carry = lax.cond(t1 < NSTAGE, odd, lambda cr: cr, carry)
                return carry

            cnt, kl16 = lax.fori_loop(0, (NSTAGE + 1) // 2, estage_pair,
                                      (jnp.int32(0), zero16))
            if p == 0:
                klv[...] = kl16
                pltpu.sync_copy(klv, kls.at[s])

            m = cnt
            ngroups = (m + _BLK - 1) // _BLK
            mround = ngroups * _BLK
            mfloor = pl.multiple_of((m // 16) * 16, 16)

            def padv(t, carry):
                base = mfloor + t * 16

                @pl.when(base < mround)
                def _():
                    sl = pl.ds(base, 16)
                    lm = (lanes + base) >= m
                    csrc_tmp[sl] = jnp.where(lm, 0, csrc_tmp[sl])
                    cdl_tmp[sl] = jnp.where(lm, trash, cdl_tmp[sl])
                    cw[sl] = jnp.where(lm, 0.0, cw[sl])
                return carry

            lax.fori_loop(0, 9, padv, 0)

            def cpy(i, carry):
                g = i // 8
                o = pl.multiple_of((i % 8) * 16, 16)
                sl = pl.ds(pl.multiple_of(i * 16, 16), 16)
                csrc2[g, pl.ds(o, 16)] = csrc_tmp[sl]
                cdl2[g, pl.ds(o, 16)] = cdl_tmp[sl]
                return carry

            lax.fori_loop(0, ngroups * 8, cpy, 0)

            def g_start(g, rows, sem):
                pltpu.async_copy(x_hbm.at[csrc2.at[g]], rows, sem)

            def g_wait(rows, sem):
                pltpu.make_async_copy(x_hbm.at[csrc2.at[0]], rows,
                                      sem).wait()

            def s_start(g, rows, sem):
                pltpu.async_copy(rows, acc.at[cdl2.at[g]], sem, add=True)

            def s_wait(rows, sem):
                pltpu.make_async_copy(rows, acc.at[cdl2.at[0]], sem).wait()

            def scale(g, rows):
                goff = pl.multiple_of(g * _BLK, _BLK)

                def scale_body(r, carry):
                    wbase = pl.multiple_of((r // 16) * 16, 16)
                    w16 = cw[pl.ds(goff + wbase, 16)]
                    wb = _lane_bcast(w16, r - wbase)
                    for j in range(_NV):
                        sj = pl.ds(j * 16, 16)
                        rows[r, sj] = rows[r, sj] * wb
                    return carry

                lax.fori_loop(0, _BLK, scale_body, 0)

            @pl.when(ngroups > 0)
            def _():
                g_start(0, rowsA, semGA)

            def gpair(pp, carry):
                g0 = 2 * pp
                g1 = 2 * pp + 1

                @pl.when(g0 < ngroups)
                def _():
                    @pl.when(g1 < ngroups)
                    def _():
                        @pl.when(g1 >= 2)
                        def _():
                            s_wait(rowsB, semSB)
                        g_start(g1, rowsB, semGB)
                    g_wait(rowsA, semGA)
                    scale(g0, rowsA)
                    s_start(g0, rowsA, semSA)

                @pl.when(g1 < ngroups)
                def _():
                    @pl.when(g1 + 1 < ngroups)
                    def _():
                        @pl.when(g1 + 1 >= 2)
                        def _():
                            s_wait(rowsA, semSA)
                        g_start(g1 + 1, rowsA, semGA)
                    g_wait(rowsB, semGB)
                    scale(g1, rowsB)
                    s_start(g1, rowsB, semSB)
                return carry

            lax.fori_loop(0, (NG + 1) // 2, gpair, 0)
            odd = (ngroups % 2) == 1

            @pl.when((ngroups >= 1) & odd)
            def _():
                s_wait(rowsA, semSA)

            @pl.when((ngroups >= 1) & ~odd)
            def _():
                s_wait(rowsB, semSB)

            @pl.when((ngroups >= 2) & odd)
            def _():
                s_wait(rowsB, semSB)

            @pl.when((ngroups >= 2) & ~odd)
            def _():
                s_wait(rowsA, semSA)

            plsc.subcore_barrier()

            def bnchunk(i, carry):
                ch = s + i * _NSUB

                @pl.when(ch < cchunks)
                def _():
                    lo = pl.multiple_of(ch * 16, 16)
                    glo = pl.multiple_of(pbase + ch * 16, 16)
                    pltpu.sync_copy(acc.at[pl.ds(lo, 16)], rb)
                    if affine:
                        pltpu.sync_copy(gamma_hbm.at[pl.ds(glo, 16)], g16)
                        pltpu.sync_copy(beta_hbm.at[pl.ds(glo, 16)], b16)

                    def row_body(r, carry2):
                        xs = [rb[r, pl.ds(j * 16, 16)] for j in range(_NV)]
                        xs[12] = jnp.where(m8, xs[12], 0.0)
                        s1 = xs[0]
                        s2 = xs[0] * xs[0]
                        for j in range(1, _NV):
                            s1 = s1 + xs[j]
                            s2 = s2 + xs[j] * xs[j]
                        t1 = jnp.sum(s1)
                        t2 = jnp.sum(s2)
                        mm = t1 * inv_b
                        var = t2 * inv_b - mm * mm
                        inv = _rsqrt16(jnp.full((16,), var + 1e-5,
                                                jnp.float32))
                        m16 = jnp.full((16,), mm, jnp.float32)
                        if affine:
                            ga = _lane_bcast(g16[...], r)
                            be = _lane_bcast(b16[...], r)
                            sc = inv * ga
                            off = be - m16 * sc
                        else:
                            sc = inv
                            off = -(m16 * inv)
                        for j in range(_NV):
                            y = xs[j] * sc + off
                            if affine:
                                y = jnp.maximum(y, 0.0)
                            if j == 12:
                                y = jnp.where(m8, y, 0.0)
                            ob[r, pl.ds(j * 16, 16)] = y
                        return carry2

                    lax.fori_loop(0, 16, row_body, 0)
                    pltpu.sync_copy(ob, y_hbm.at[pl.ds(glo, 16)])
                return carry

            lax.fori_loop(0, NMAX_BN, bnchunk, 0)

            if p + 1 < NPASS:
                for r in range(16):
                    for j in range(_NV):
                        rb[r, pl.ds(j * 16, 16)] = zero16
                plsc.subcore_barrier()

        @pl.when((c == 0) & (s == 0))
        def _():
            pltpu.sync_copy(kls, klbuf)
            tot = klbuf[0, pl.ds(0, 16)]
            for r in range(1, _NSUB):
                tot = tot + klbuf[r, pl.ds(0, 16)]
            klv[...] = jnp.full((16,), jnp.sum(tot), jnp.float32)
            pltpu.sync_copy(klv, kl_hbm)

    return pl.kernel(
        body, out_type, mesh=mesh, scratch_types=scratch,
        compiler_params=pltpu.CompilerParams(needs_layout_passes=False,
                                             use_tc_tiling_on_sc=False))


def _tc_transpose(x):
    n = x.shape[1]
    w = 1024
    grid = pl.cdiv(n, w)

    def tbody(x_ref, o_ref):
        xt = jnp.transpose(x_ref[...])
        o_ref[...] = jnp.concatenate(
            [xt, jnp.zeros((w, _BPAD - _B), jnp.float32)], axis=1)

    return pl.pallas_call(
        tbody,
        grid=(grid,),
        in_specs=[pl.BlockSpec((_B, w), lambda i: (0, i))],
        out_specs=pl.BlockSpec((w, _BPAD), lambda i: (i, 0)),
        out_shape=jax.ShapeDtypeStruct((n, _BPAD), jnp.float32),
    )(x)


def _pack_edges(idx, mu, lv, ep, e_pad):
    e = idx.shape[0]
    pad = e_pad - e
    srcf = lax.bitcast_convert_type(jnp.pad(idx[:, 0], (0, pad)), jnp.float32)
    dstf = lax.bitcast_convert_type(
        jnp.pad(idx[:, 1], (0, pad), constant_values=1 << 30), jnp.float32)
    packed = jnp.stack([srcf, dstf, jnp.pad(mu, (0, pad)),
                        jnp.pad(lv, (0, pad)), jnp.pad(ep, (0, pad))], axis=0)
    return packed.reshape(5, e_pad // _BLK, _BLK).transpose(1, 0, 2)


def _specs():
    specs = []
    n_in_rows = _NODES[0]
    for i in range(5):
        n_out = _NODES[i + 1]
        e_pad = -(-_EDGES[i] // (_NSUB * _BLK)) * (_NSUB * _BLK)
        nblk_tile = e_pad // (_NSUB * _BLK)
        affine = i < 4
        H0 = ((n_out + 31) // 32) * 16
        H1 = n_out - H0
        npad_out = H0 + ((H1 + 15) // 16) * 16
        specs.append(dict(n_in_rows=n_in_rows, n_out=n_out, e_pad=e_pad,
                          nblk_tile=nblk_tile, affine=affine,
                          npad_out=npad_out))
        n_in_rows = npad_out
    return specs


_SPECS = _specs()
_KERNELS = [_make_level(sp["n_in_rows"], sp["n_out"], sp["nblk_tile"],
                        sp["affine"]) for sp in _SPECS]


def kernel(x, idx1, mu1, logvar1, bias1, eps1, idx2, mu2, logvar2, bias2,
           eps2, idx3, mu3, logvar3, bias3, eps3, idx4, mu4, logvar4, bias4,
           eps4, idx5, mu5, logvar5, bias5, eps5, g1, b1, g2, b2, g3, b3,
           g4, b4):
    idxs = [idx1, idx2, idx3, idx4, idx5]
    mus = [mu1, mu2, mu3, mu4, mu5]
    lvs = [logvar1, logvar2, logvar3, logvar4, logvar5]
    eps = [eps1, eps2, eps3, eps4, eps5]
    gammas = [g1, g2, g3, g4, None]
    betas = [b1, b2, b3, b4, None]

    cur = _tc_transpose(x.reshape(_B, -1))
    kl_tot = jnp.float32(0.0)
    for i in range(5):
        sp = _SPECS[i]
        packed = _pack_edges(idxs[i], mus[i], lvs[i], eps[i], sp["e_pad"])
        if sp["affine"]:
            gp = jnp.pad(gammas[i], (0, sp["npad_out"] - sp["n_out"]))
            bp = jnp.pad(betas[i], (0, sp["npad_out"] - sp["n_out"]))
            cur, kl16 = _KERNELS[i](cur, packed, gp, bp)
        else:
            cur, kl16 = _KERNELS[i](cur, packed)
        kl_tot = kl_tot + kl16[0]
    y = jnp.transpose(cur[:, :_B])
    return y, kl_tot


# speedup vs baseline: 2.5521x; 1.0264x over previous
import jax
import jax.numpy as jnp
from jax import lax
from jax.experimental import pallas as pl
from jax.experimental.pallas import tpu as pltpu
from jax.experimental.pallas import tpu_sc as plsc

_B = 200
_BPAD = 208
_NV = _BPAD // 16
_BLK = 128
_NSUB = 16
_NODES = [30000, 10000, 2000, 500, 100, 32]
_EDGES = [90000, 60000, 12000, 2000, 400]


def _lane_bcast(v16, r):
    idx = jnp.full((16,), r, dtype=jnp.int32)
    return jnp.take_along_axis(v16, idx, axis=0)


def _rsqrt16(v):
    i = lax.bitcast_convert_type(v, jnp.int32)
    y = lax.bitcast_convert_type(
        jnp.int32(0x5F3759DF) - lax.shift_right_logical(i, 1), jnp.float32)
    for _ in range(3):
        y = y * (1.5 - 0.5 * v * y * y)
    return y


def _mask8():
    return jnp.arange(16, dtype=jnp.int32) < 8


def _stg(nblk_tile):
    for d in (8, 7, 6, 5, 4, 3, 2, 1):
        if nblk_tile % d == 0:
            return d
    return 1


def _make_level(n_in_rows, n_out, nblk_tile, affine):
    H0 = ((n_out + 31) // 32) * 16
    H1 = n_out - H0
    NPASS = 2 if n_out >= 4000 else 1
    Q = -(-H0 // (16 * NPASS)) * 16
    ACCR = Q + 16
    NPAD_OUT = H0 + ((H1 + 15) // 16) * 16
    NMAX_BN = (Q // 16 + _NSUB - 1) // _NSUB
    CZ = ACCR // 16
    NMAX_Z = (CZ + _NSUB - 1) // _NSUB
    STG = _stg(nblk_tile)
    NSTAGE = nblk_tile // STG
    NG = nblk_tile
    CAP = NG * _BLK + 16

    mesh = plsc.VectorSubcoreMesh(core_axis_name="c", subcore_axis_name="s")
    out_type = (jax.ShapeDtypeStruct((NPAD_OUT, _BPAD), jnp.float32),
                jax.ShapeDtypeStruct((16,), jnp.float32))
    scratch = [
        pltpu.VMEM_SHARED((ACCR, _BPAD), jnp.float32),
        pltpu.VMEM_SHARED((_NSUB, 16), jnp.float32),
        pltpu.VMEM((STG, 5, _BLK), jnp.float32),
        pltpu.VMEM((STG, 5, _BLK), jnp.float32),
        pltpu.VMEM((CAP,), jnp.int32),
        pltpu.VMEM((CAP,), jnp.int32),
        pltpu.VMEM((CAP,), jnp.float32),
        pltpu.VMEM((NG, _BLK), jnp.int32),
        pltpu.VMEM((NG, _BLK), jnp.int32),
        pltpu.VMEM((_BLK, _BPAD), jnp.float32),
        pltpu.VMEM((_BLK, _BPAD), jnp.float32),
        pltpu.VMEM((16, _BPAD), jnp.float32),
        pltpu.VMEM((16, _BPAD), jnp.float32),
        pltpu.VMEM((16,), jnp.float32),
        pltpu.VMEM((16,), jnp.float32),
        pltpu.VMEM((16,), jnp.float32),
        pltpu.VMEM((_NSUB, 16), jnp.float32),
        pltpu.SemaphoreType.DMA,
        pltpu.SemaphoreType.DMA,
        pltpu.SemaphoreType.DMA,
        pltpu.SemaphoreType.DMA,
        pltpu.SemaphoreType.DMA,
        pltpu.SemaphoreType.DMA,
    ]

    def body(*refs):
        if affine:
            (x_hbm, edges_hbm, gamma_hbm, beta_hbm, y_hbm, kl_hbm,
             acc, kls, ebufA, ebufB, csrc_tmp, cdl_tmp, cw, csrc2, cdl2,
             rowsA, rowsB, rb, ob, g16, b16, klv, klbuf,
             semEA, semEB, semGA, semGB, semSA, semSB) = refs
        else:
            (x_hbm, edges_hbm, y_hbm, kl_hbm,
             acc, kls, ebufA, ebufB, csrc_tmp, cdl_tmp, cw, csrc2, cdl2,
             rowsA, rowsB, rb, ob, g16, b16, klv, klbuf,
             semEA, semEB, semGA, semGB, semSA, semSB) = refs
        c = lax.axis_index("c")
        s = lax.axis_index("s")
        cbase = c * H0
        hc = jnp.where(c == 0, H0, H1).astype(jnp.int32)
        zero16 = jnp.zeros((16,), jnp.float32)
        lanes = jnp.arange(16, dtype=jnp.int32)
        tbase = s * nblk_tile
        inv_b = jnp.float32(1.0 / _B)
        m8 = _mask8()

        for r in range(16):
            for j in range(_NV):
                rb[r, pl.ds(j * 16, 16)] = zero16

        for p in range(NPASS):
            pbase = cbase + p * Q
            hcp = jnp.clip(hc - p * Q, 0, Q)
            trash = hcp + s
            cchunks = (hcp + 15) // 16

            def zbody(i, carry):
                ch = s + i * _NSUB

                @pl.when(ch < CZ)
                def _():
                    pltpu.async_copy(
                        rb, acc.at[pl.ds(pl.multiple_of(ch * 16, 16), 16)],
                        semGA)
                return carry

            lax.fori_loop(0, NMAX_Z, zbody, 0)

            def zdrain(i, carry):
                ch = s + i * _NSUB

                @pl.when(ch < CZ)
                def _():
                    pltpu.make_async_copy(
                        rb, acc.at[pl.ds(pl.multiple_of(ch * 16, 16), 16)],
                        semGA).wait()
                return carry

            lax.fori_loop(0, NMAX_Z, zdrain, 0)
            plsc.subcore_barrier()

            def estage_start(t, ebuf, sem):
                src = edges_hbm.at[pl.ds(tbase + t * STG, STG)]
                pltpu.async_copy(src, ebuf, sem)

            def estage_wait(ebuf, sem):
                pltpu.make_async_copy(edges_hbm.at[pl.ds(0, STG)], ebuf,
                                      sem).wait()

            def estage_proc(ebuf, carry):
                cnt, kl16 = carry
                for q in range(STG):
                    for k in range(8):
                        sl = pl.ds(k * 16, 16)
                        src_i = lax.bitcast_convert_type(ebuf[q, 0, sl],
                                                         jnp.int32)
                        d = lax.bitcast_convert_type(ebuf[q, 1, sl],
                                                     jnp.int32)
                        dl = d - pbase
                        ok = (dl >= 0) & (dl < hcp)
                        lv = ebuf[q, 3, sl]
                        mu = ebuf[q, 2, sl]
                        eh = jnp.exp(0.5 * lv)
                        w = mu + eh * ebuf[q, 4, sl]
                        if p == 0:
                            kl16 = kl16 + 0.5 * (mu * mu + eh * eh - lv
                                                 - 1.0)
                        plsc.store_compressed(csrc_tmp.at[pl.ds(cnt, 16)],
                                              src_i, mask=ok)
                        plsc.store_compressed(cdl_tmp.at[pl.ds(cnt, 16)],
                                              dl, mask=ok)
                        plsc.store_compressed(cw.at[pl.ds(cnt, 16)], w,
                                              mask=ok)
                        cnt = cnt + jnp.sum(
                            jnp.where(ok, 1.0, 0.0)).astype(jnp.int32)
                return cnt, kl16

            estage_start(0, ebufA, semEA)

            def estage_pair(pp, carry):
                t1 = 2 * pp + 1

                @pl.when(t1 < NSTAGE)
                def _():
                    estage_start(t1, ebufB, semEB)
                estage_wait(ebufA, semEA)
                carry = estage_proc(ebufA, carry)

                def odd(carry):
                    @pl.when(t1 + 1 < NSTAGE)
                    def _():
                        estage_start(t1 + 1, ebufA, semEA)
                    estage_wait(ebufB, semEB)
                    return estage_proc(ebufB, carry)

                carry = lax.cond(t1 < NSTAGE, odd, lambda cr: cr, carry)
                return carry

            cnt, kl16 = lax.fori_loop(0, (NSTAGE + 1) // 2, estage_pair,
                                      (jnp.int32(0), zero16))
            if p == 0:
                klv[...] = kl16
                pltpu.sync_copy(klv, kls.at[s])

            m = cnt
            ngroups = (m + _BLK - 1) // _BLK
            mround = ngroups * _BLK
            mfloor = pl.multiple_of((m // 16) * 16, 16)

            def padv(t, carry):
                base = mfloor + t * 16

                @pl.when(base < mround)
                def _():
                    sl = pl.ds(base, 16)
                    lm = (lanes + base) >= m
                    csrc_tmp[sl] = jnp.where(lm, 0, csrc_tmp[sl])
                    cdl_tmp[sl] = jnp.where(lm, trash, cdl_tmp[sl])
                    cw[sl] = jnp.where(lm, 0.0, cw[sl])
                return carry

            lax.fori_loop(0, 9, padv, 0)

            def cpy(i, carry):
                g = i // 8
                o = pl.multiple_of((i % 8) * 16, 16)
                sl = pl.ds(pl.multiple_of(i * 16, 16), 16)
                csrc2[g, pl.ds(o, 16)] = csrc_tmp[sl]
                cdl2[g, pl.ds(o, 16)] = cdl_tmp[sl]
                return carry

            lax.fori_loop(0, ngroups * 8, cpy, 0)

            def g_start(g, rows, sem):
                pltpu.async_copy(x_hbm.at[csrc2.at[g]], rows, sem)

            def g_wait(rows, sem):
                pltpu.make_async_copy(x_hbm.at[csrc2.at[0]], rows,
                                      sem).wait()

            def s_start(g, rows, sem):
                pltpu.async_copy(rows, acc.at[cdl2.at[g]], sem, add=True)

            def s_wait(rows, sem):
                pltpu.make_async_copy(rows, acc.at[cdl2.at[0]], sem).wait()

            def scale(g, rows):
                goff = pl.multiple_of(g * _BLK, _BLK)

                def scale_body(r, carry):
                    wbase = pl.multiple_of((r // 16) * 16, 16)
                    w16 = cw[pl.ds(goff + wbase, 16)]
                    wb = _lane_bcast(w16, r - wbase)
                    for j in range(_NV):
                        sj = pl.ds(j * 16, 16)
                        rows[r, sj] = rows[r, sj] * wb
                    return carry

                lax.fori_loop(0, _BLK, scale_body, 0)

            @pl.when(ngroups > 0)
            def _():
                g_start(0, rowsA, semGA)

            def gpair(pp, carry):
                g0 = 2 * pp
                g1 = 2 * pp + 1

                @pl.when(g0 < ngroups)
                def _():
                    @pl.when(g1 < ngroups)
                    def _():
                        @pl.when(g1 >= 2)
                        def _():
                            s_wait(rowsB, semSB)
                        g_start(g1, rowsB, semGB)
                    g_wait(rowsA, semGA)
                    scale(g0, rowsA)
                    s_start(g0, rowsA, semSA)

                @pl.when(g1 < ngroups)
                def _():
                    @pl.when(g1 + 1 < ngroups)
                    def _():
                        @pl.when(g1 + 1 >= 2)
                        def _():
                            s_wait(rowsA, semSA)
                        g_start(g1 + 1, rowsA, semGA)
                    g_wait(rowsB, semGB)
                    scale(g1, rowsB)
                    s_start(g1, rowsB, semSB)
                return carry

            lax.fori_loop(0, (NG + 1) // 2, gpair, 0)
            odd = (ngroups % 2) == 1

            @pl.when((ngroups >= 1) & odd)
            def _():
                s_wait(rowsA, semSA)

            @pl.when((ngroups >= 1) & ~odd)
            def _():
                s_wait(rowsB, semSB)

            @pl.when((ngroups >= 2) & odd)
            def _():
                s_wait(rowsB, semSB)

            @pl.when((ngroups >= 2) & ~odd)
            def _():
                s_wait(rowsA, semSA)

            plsc.subcore_barrier()

            def bnchunk(i, carry):
                ch = s + i * _NSUB

                @pl.when(ch < cchunks)
                def _():
                    lo = pl.multiple_of(ch * 16, 16)
                    glo = pl.multiple_of(pbase + ch * 16, 16)
                    pltpu.async_copy(acc.at[pl.ds(lo, 16)], rb, semGA)
                    if affine:
                        pltpu.async_copy(gamma_hbm.at[pl.ds(glo, 16)], g16,
                                         semGB)
                        pltpu.async_copy(beta_hbm.at[pl.ds(glo, 16)], b16,
                                         semGB)
                    pltpu.make_async_copy(acc.at[pl.ds(lo, 16)], rb,
                                          semGA).wait()
                    if affine:
                        pltpu.make_async_copy(gamma_hbm.at[pl.ds(glo, 16)],
                                              g16, semGB).wait()
                        pltpu.make_async_copy(beta_hbm.at[pl.ds(glo, 16)],
                                              b16, semGB).wait()

                    def row_body(r, carry2):
                        xs = [rb[r, pl.ds(j * 16, 16)] for j in range(_NV)]
                        xs[12] = jnp.where(m8, xs[12], 0.0)
                        s1 = xs[0]
                        s2 = xs[0] * xs[0]
                        for j in range(1, _NV):
                            s1 = s1 + xs[j]
                            s2 = s2 + xs[j] * xs[j]
                        t1 = jnp.sum(s1)
                        t2 = jnp.sum(s2)
                        mm = t1 * inv_b
                        var = t2 * inv_b - mm * mm
                        inv = _rsqrt16(jnp.full((16,), var + 1e-5,
                                                jnp.float32))
                        m16 = jnp.full((16,), mm, jnp.float32)
                        if affine:
                            ga = _lane_bcast(g16[...], r)
                            be = _lane_bcast(b16[...], r)
                            sc = inv * ga
                            off = be - m16 * sc
                        else:
                            sc = inv
                            off = -(m16 * inv)
                        for j in range(_NV):
                            y = xs[j] * sc + off
                            if affine:
                                y = jnp.maximum(y, 0.0)
                            if j == 12:
                                y = jnp.where(m8, y, 0.0)
                            ob[r, pl.ds(j * 16, 16)] = y
                        return carry2

                    lax.fori_loop(0, 16, row_body, 0)
                    pltpu.sync_copy(ob, y_hbm.at[pl.ds(glo, 16)])
                return carry

            lax.fori_loop(0, NMAX_BN, bnchunk, 0)

            if p + 1 < NPASS:
                for r in range(16):
                    for j in range(_NV):
                        rb[r, pl.ds(j * 16, 16)] = zero16
                plsc.subcore_barrier()

        @pl.when((c == 0) & (s == 0))
        def _():
            pltpu.sync_copy(kls, klbuf)
            tot = klbuf[0, pl.ds(0, 16)]
            for r in range(1, _NSUB):
                tot = tot + klbuf[r, pl.ds(0, 16)]
            klv[...] = jnp.full((16,), jnp.sum(tot), jnp.float32)
            pltpu.sync_copy(klv, kl_hbm)

    return pl.kernel(
        body, out_type, mesh=mesh, scratch_types=scratch,
        compiler_params=pltpu.CompilerParams(needs_layout_passes=False,
                                             use_tc_tiling_on_sc=False))


def _tc_transpose(x):
    n = x.shape[1]
    w = 1024
    grid = pl.cdiv(n, w)

    def tbody(x_ref, o_ref):
        xt = jnp.transpose(x_ref[...])
        o_ref[...] = jnp.concatenate(
            [xt, jnp.zeros((w, _BPAD - _B), jnp.float32)], axis=1)

    return pl.pallas_call(
        tbody,
        grid=(grid,),
        in_specs=[pl.BlockSpec((_B, w), lambda i: (0, i))],
        out_specs=pl.BlockSpec((w, _BPAD), lambda i: (i, 0)),
        out_shape=jax.ShapeDtypeStruct((n, _BPAD), jnp.float32),
    )(x)


def _pack_edges(idx, mu, lv, ep, e_pad):
    e = idx.shape[0]
    pad = e_pad - e
    srcf = lax.bitcast_convert_type(jnp.pad(idx[:, 0], (0, pad)), jnp.float32)
    dstf = lax.bitcast_convert_type(
        jnp.pad(idx[:, 1], (0, pad), constant_values=1 << 30), jnp.float32)
    packed = jnp.stack([srcf, dstf, jnp.pad(mu, (0, pad)),
                        jnp.pad(lv, (0, pad)), jnp.pad(ep, (0, pad))], axis=0)
    return packed.reshape(5, e_pad // _BLK, _BLK).transpose(1, 0, 2)


def _specs():
    specs = []
    n_in_rows = _NODES[0]
    for i in range(5):
        n_out = _NODES[i + 1]
        e_pad = -(-_EDGES[i] // (_NSUB * _BLK)) * (_NSUB * _BLK)
        nblk_tile = e_pad // (_NSUB * _BLK)
        affine = i < 4
        H0 = ((n_out + 31) // 32) * 16
        H1 = n_out - H0
        npad_out = H0 + ((H1 + 15) // 16) * 16
        specs.append(dict(n_in_rows=n_in_rows, n_out=n_out, e_pad=e_pad,
                          nblk_tile=nblk_tile, affine=affine,
                          npad_out=npad_out))
        n_in_rows = npad_out
    return specs


_SPECS = _specs()
_KERNELS = [_make_level(sp["n_in_rows"], sp["n_out"], sp["nblk_tile"],
                        sp["affine"]) for sp in _SPECS]


def kernel(x, idx1, mu1, logvar1, bias1, eps1, idx2, mu2, logvar2, bias2,
           eps2, idx3, mu3, logvar3, bias3, eps3, idx4, mu4, logvar4, bias4,
           eps4, idx5, mu5, logvar5, bias5, eps5, g1, b1, g2, b2, g3, b3,
           g4, b4):
    idxs = [idx1, idx2, idx3, idx4, idx5]
    mus = [mu1, mu2, mu3, mu4, mu5]
    lvs = [logvar1, logvar2, logvar3, logvar4, logvar5]
    eps = [eps1, eps2, eps3, eps4, eps5]
    gammas = [g1, g2, g3, g4, None]
    betas = [b1, b2, b3, b4, None]

    cur = _tc_transpose(x.reshape(_B, -1))
    kl_tot = jnp.float32(0.0)
    for i in range(5):
        sp = _SPECS[i]
        packed = _pack_edges(idxs[i], mus[i], lvs[i], eps[i], sp["e_pad"])
        if sp["affine"]:
            gp = jnp.pad(gammas[i], (0, sp["npad_out"] - sp["n_out"]))
            bp = jnp.pad(betas[i], (0, sp["npad_out"] - sp["n_out"]))
            cur, kl16 = _KERNELS[i](cur, packed, gp, bp)
        else:
            cur, kl16 = _KERNELS[i](cur, packed)
        kl_tot = kl_tot + kl16[0]
    y = jnp.transpose(cur[:, :_B])
    return y, kl_tot
